# Initial kernel scaffold; baseline (speedup 1.0000x reference)
import jax
import jax.numpy as jnp
from jax.experimental import pallas as pl


def kernel(h0, x, all_edges, all_edge_attr, n_nodes, emb_W, emb_b, gcn_W, gcn_b, dec_W1, dec_b1, dec_W2, dec_b2, gd_W1, gd_b1, gd_W2, gd_b2):
    raise NotImplementedError("write your pallas kernel here")


# speedup vs baseline: 1.0173x; 1.0173x over previous
import jax
import jax.numpy as jnp
from jax.experimental import pallas as pl

N_GRAPHS = 2
N_LAYERS = 8
N = 10000
H = 512


def _gcn_conv(h, src, dst, W, b):
    loop = jnp.arange(N, dtype=src.dtype)
    s = jnp.concatenate([src, loop])
    d = jnp.concatenate([dst, loop])
    deg = jax.ops.segment_sum(jnp.ones(s.shape[0], dtype=h.dtype), d, num_segments=N)
    dinv = jnp.where(deg > 0, jax.lax.rsqrt(jnp.maximum(deg, 1e-12)), 0.0)
    norm = dinv[s] * dinv[d]
    xw = h @ W
    out = jax.ops.segment_sum(xw[s] * norm[:, None], d, num_segments=N)
    return out + b


def _final_kernel(combined_ref, w1_ref, b1_ref, w2_ref, b2_ref, out_ref):
    t = jnp.maximum(
        jax.lax.dot(combined_ref[...], w1_ref[...],
                    precision=jax.lax.Precision.HIGHEST) + b1_ref[...], 0.0)
    out_ref[...] = jax.lax.dot(t, w2_ref[...],
                               precision=jax.lax.Precision.HIGHEST) + b2_ref[...]


def kernel(h0, x, all_edges, all_edge_attr, n_nodes, emb_W, emb_b, gcn_W, gcn_b, dec_W1, dec_b1, dec_W2, dec_b2, gd_W1, gd_b1, gd_W2, gd_b2):
    hf = []
    for j in range(N_GRAPHS):
        h = h0[j] @ emb_W + emb_b
        src = all_edges[j, 0]
        dst = all_edges[j, 1]
        for i in range(N_LAYERS):
            h = _gcn_conv(h, src, dst, gcn_W[j, i], gcn_b[j, i])
        h = jax.nn.relu(h @ dec_W1[j] + dec_b1[j]) @ dec_W2[j] + dec_b2[j]
        hf.append(jnp.sum(h, axis=0, keepdims=True))
    combined = jnp.concatenate(hf, axis=1)
    pred = pl.pallas_call(
        _final_kernel,
        out_shape=jax.ShapeDtypeStruct((1, 1), jnp.float32),
    )(combined, gd_W1, gd_b1[None, :], gd_W2, gd_b2[None, :])
    return pred[:, 0]


# speedup vs baseline: 3.9356x; 3.8686x over previous
import jax
import jax.numpy as jnp
from jax import lax
from jax.experimental import pallas as pl
from jax.experimental.pallas import tpu as pltpu
from jax.experimental.pallas import tpu_sc as plsc

N_GRAPHS = 2
N_LAYERS = 8
N = 10000
E = 160000
IN_NF = 256
H = 512

NC = 2
NS = 16
HS = 64
NHS = H // HS
CH = 128
NBUF = 4
EPW = 10240
NCHUNK = EPW // CH
NGROUP = NCHUNK // NBUF
EP = EPW * NS
ZROWS = 10112
RPW = ZROWS // NS
SINK = N

_HIGHEST = jax.lax.Precision.HIGHEST
_mesh = plsc.VectorSubcoreMesh(core_axis_name="c", subcore_axis_name="s")
_SC_PARAMS = pltpu.CompilerParams(use_tc_tiling_on_sc=False)


def _deg_body(dst3, out, idx_v, ones_v, zero_v, sem, deg_sp):
    c = lax.axis_index("c")
    s = lax.axis_index("s")
    half = NCHUNK // 2
    for i in range(RPW // 16 + 1):
        zero_v[pl.ds(16 * i, 16)] = jnp.zeros((16,), jnp.float32)
    for i in range(CH // 16):
        ones_v[pl.ds(16 * i, 16)] = jnp.full((16,), 1.0, jnp.float32)
    pltpu.sync_copy(zero_v.at[pl.ds(0, RPW)], deg_sp.at[pl.ds(s * RPW, RPW)])
    pltpu.sync_copy(dst3.at[s], idx_v)
    plsc.subcore_barrier()

    def chunk(j, _):
        jj = j + c * half
        pltpu.async_copy(ones_v, deg_sp.at[idx_v.at[jj]], sem, add=True).wait()
        return 0

    lax.fori_loop(0, half, chunk, 0)
    plsc.subcore_barrier()
    off = pl.multiple_of(c * ZROWS + s * RPW, 8)
    pltpu.sync_copy(deg_sp.at[pl.ds(s * RPW, RPW)], zero_v.at[pl.ds(0, RPW)])
    pltpu.sync_copy(zero_v.at[pl.ds(0, RPW)], out.at[pl.ds(off, RPW)])


def _make_deg_kernel():
    return pl.kernel(
        _deg_body,
        out_type=jax.ShapeDtypeStruct((NC * ZROWS,), jnp.float32),
        mesh=_mesh,
        scratch_types=[
            pltpu.VMEM((NCHUNK, CH), jnp.int32),
            pltpu.VMEM((CH,), jnp.float32),
            pltpu.VMEM((RPW + 16,), jnp.float32),
            pltpu.SemaphoreType.DMA,
            pltpu.VMEM_SHARED((ZROWS,), jnp.float32),
        ],
        compiler_params=_SC_PARAMS,
    )


def _scatter_body(*args):
    ys = args[:NHS]
    src3, dst3 = args[NHS], args[NHS + 1]
    zouts = args[NHS + 2:2 * NHS + 2]
    src_v, dst_v, gbuf, sem_g, sem_s, z_sp = args[2 * NHS + 2:]
    c = lax.axis_index("c")
    s = lax.axis_index("s")
    pltpu.sync_copy(src3.at[s], src_v)
    pltpu.sync_copy(dst3.at[s], dst_v)
    row0 = pl.multiple_of(s * RPW, 8)
    nlast = N - (NS - 1) * RPW

    def _stage(nrows, y, zout, to_spmem):
        r = 0
        while r < nrows:
            nr = min(CH, nrows - r)
            rows = pl.ds(row0 + r, nr)
            stg = gbuf.at[0, pl.ds(0, nr)]
            if to_spmem:
                pltpu.sync_copy(y.at[rows], stg)
                pltpu.sync_copy(stg, z_sp.at[rows])
            else:
                pltpu.sync_copy(z_sp.at[rows], stg)
                pltpu.sync_copy(stg, zout.at[rows])
            r += nr

    for p in range(NHS):
        y, zout = ys[p], zouts[p]

        @pl.when(c == p // (NHS // NC))
        def _pass():
            @pl.when(s < NS - 1)
            def _():
                _stage(RPW, y, zout, True)
            @pl.when(s == NS - 1)
            def _():
                _stage(nlast, y, zout, True)
            plsc.subcore_barrier()

            for b in range(NBUF):
                pltpu.async_copy(y.at[src_v.at[b]], gbuf.at[b], sem_g[b])

            def group(g, _):
                for b in range(NBUF):
                    j = g * NBUF + b
                    pltpu.make_async_copy(y.at[src_v.at[j]], gbuf.at[b],
                                          sem_g[b]).wait()
                    pltpu.async_copy(gbuf.at[b], z_sp.at[dst_v.at[j]],
                                     sem_s[b], add=True)
                for b in range(NBUF):
                    j = g * NBUF + b
                    pltpu.make_async_copy(gbuf.at[b], z_sp.at[dst_v.at[j]],
                                          sem_s[b]).wait()
                    @pl.when(g < NGROUP - 1)
                    def _():
                        pltpu.async_copy(y.at[src_v.at[j + NBUF]],
                                         gbuf.at[b], sem_g[b])
                return 0

            lax.fori_loop(0, NGROUP, group, 0)
            plsc.subcore_barrier()
            @pl.when(s < NS - 1)
            def _():
                _stage(RPW, y, zout, False)
            @pl.when(s == NS - 1)
            def _():
                _stage(nlast, y, zout, False)
            plsc.subcore_barrier()


def _make_scatter_kernel():
    return pl.kernel(
        _scatter_body,
        out_type=[jax.ShapeDtypeStruct((N, HS), jnp.float32)] * NHS,
        mesh=_mesh,
        scratch_types=[
            pltpu.VMEM((NCHUNK, CH), jnp.int32),
            pltpu.VMEM((NCHUNK, CH), jnp.int32),
            pltpu.VMEM((NBUF, CH, HS), jnp.float32),
            [pltpu.SemaphoreType.DMA] * NBUF,
            [pltpu.SemaphoreType.DMA] * NBUF,
            pltpu.VMEM_SHARED((ZROWS, HS), jnp.float32),
        ],
        compiler_params=_SC_PARAMS,
    )


_BR = 1000
_GRID = N // _BR

_ZSPEC = [pl.BlockSpec((_BR, HS), lambda r: (r, 0))] * NHS
_YSPEC = [pl.BlockSpec((_BR, HS), lambda r: (r, 0))] * NHS
_YSHAPE = [jax.ShapeDtypeStruct((N, HS), jnp.float32)] * NHS


def _emb_y_kernel(h0_ref, embw_ref, embb_ref, w_ref, dinv_ref, *y_refs):
    h = jax.lax.dot(h0_ref[...], embw_ref[...], precision=_HIGHEST) + embb_ref[...]
    y = dinv_ref[...] * jax.lax.dot(h, w_ref[...], precision=_HIGHEST)
    for p in range(NHS):
        y_refs[p][...] = y[:, p * HS:(p + 1) * HS]


def _emb_y(h0, emb_W, emb_b, W0, dinv):
    return pl.pallas_call(
        _emb_y_kernel,
        grid=(_GRID,),
        in_specs=[
            pl.BlockSpec((_BR, IN_NF), lambda r: (r, 0)),
            pl.BlockSpec((IN_NF, H), lambda r: (0, 0)),
            pl.BlockSpec((1, H), lambda r: (0, 0)),
            pl.BlockSpec((H, H), lambda r: (0, 0)),
            pl.BlockSpec((_BR, 1), lambda r: (r, 0)),
        ],
        out_specs=_YSPEC,
        out_shape=_YSHAPE,
    )(h0, emb_W, emb_b, W0, dinv)


def _y_kernel(*refs):
    z_refs = refs[:NHS]
    dinv_ref, b_ref, w_ref = refs[NHS:NHS + 3]
    y_refs = refs[NHS + 3:]
    dinv = dinv_ref[...]
    z = jnp.concatenate([zr[...] for zr in z_refs], axis=1)
    h = dinv * z + b_ref[...]
    y = dinv * jax.lax.dot(h, w_ref[...], precision=_HIGHEST)
    for p in range(NHS):
        y_refs[p][...] = y[:, p * HS:(p + 1) * HS]


def _y_step(zs, dinv, b, W):
    return pl.pallas_call(
        _y_kernel,
        grid=(_GRID,),
        in_specs=_ZSPEC + [
            pl.BlockSpec((_BR, 1), lambda r: (r, 0)),
            pl.BlockSpec((1, H), lambda r: (0, 0)),
            pl.BlockSpec((H, H), lambda r: (0, 0)),
        ],
        out_specs=_YSPEC,
        out_shape=_YSHAPE,
    )(*zs, dinv, b, W)


def _dec_kernel(*refs):
    z_refs = refs[:NHS]
    dinv_ref, b_ref, w1_ref, b1_ref, w2_ref, b2_ref, out_ref = refs[NHS:]
    dinv = dinv_ref[...]
    z = jnp.concatenate([zr[...] for zr in z_refs], axis=1)
    h = dinv * z + b_ref[...]
    t = jnp.maximum(jax.lax.dot(h, w1_ref[...], precision=_HIGHEST) + b1_ref[...],
                    0.0)
    o = jax.lax.dot(t, w2_ref[...], precision=_HIGHEST) + b2_ref[...]
    part = jnp.sum(o, axis=0, keepdims=True)

    @pl.when(pl.program_id(0) == 0)
    def _():
        out_ref[...] = jnp.zeros_like(out_ref)

    out_ref[...] += part


def _dec_sum(zs, dinv, b, W1, b1, W2, b2):
    return pl.pallas_call(
        _dec_kernel,
        grid=(_GRID,),
        in_specs=_ZSPEC + [
            pl.BlockSpec((_BR, 1), lambda r: (r, 0)),
            pl.BlockSpec((1, H), lambda r: (0, 0)),
            pl.BlockSpec((H, H), lambda r: (0, 0)),
            pl.BlockSpec((1, H), lambda r: (0, 0)),
            pl.BlockSpec((H, H), lambda r: (0, 0)),
            pl.BlockSpec((1, H), lambda r: (0, 0)),
        ],
        out_specs=pl.BlockSpec((1, H), lambda r: (0, 0)),
        out_shape=jax.ShapeDtypeStruct((1, H), jnp.float32),
    )(*zs, dinv, b, W1, b1, W2, b2)


def _gd_kernel(c_ref, w1_ref, b1_ref, w2_ref, b2_ref, out_ref):
    t = jnp.maximum(
        jax.lax.dot(c_ref[...], w1_ref[...], precision=_HIGHEST) + b1_ref[...],
        0.0)
    out_ref[...] = jax.lax.dot(t, w2_ref[...], precision=_HIGHEST) + b2_ref[...]


def kernel(h0, x, all_edges, all_edge_attr, n_nodes, emb_W, emb_b, gcn_W,
           gcn_b, dec_W1, dec_b1, dec_W2, dec_b2, gd_W1, gd_b1, gd_W2, gd_b2):
    deg_kernel = _make_deg_kernel()
    scatter_kernel = _make_scatter_kernel()
    pad_i = jnp.zeros((EP - E,), jnp.int32)
    pad_s = jnp.full((EP - E,), SINK, jnp.int32)

    sums = []
    for j in range(N_GRAPHS):
        src3 = jnp.concatenate([all_edges[j, 0], pad_i]).reshape(NS, NCHUNK, CH)
        dst3 = jnp.concatenate([all_edges[j, 1], pad_s]).reshape(NS, NCHUNK, CH)
        degp = deg_kernel(dst3)
        deg = degp[:N] + degp[ZROWS:ZROWS + N] + 1.0
        dinv = jax.lax.rsqrt(deg)[:, None]

        ys = _emb_y(h0[j], emb_W, emb_b[None, :], gcn_W[j, 0], dinv)
        for i in range(N_LAYERS):
            zs = scatter_kernel(*ys, src3, dst3)
            if i < N_LAYERS - 1:
                ys = _y_step(zs, dinv, gcn_b[j, i][None, :], gcn_W[j, i + 1])
            else:
                sums.append(_dec_sum(zs, dinv, gcn_b[j, i][None, :], dec_W1[j],
                                     dec_b1[j][None, :], dec_W2[j],
                                     dec_b2[j][None, :]))

    combined = jnp.concatenate(sums, axis=1)
    pred = pl.pallas_call(
        _gd_kernel,
        out_shape=jax.ShapeDtypeStruct((1, 1), jnp.float32),
    )(combined, gd_W1, gd_b1[None, :], gd_W2, gd_b2[None, :])
    return pred[:, 0]


# speedup vs baseline: 4.2000x; 1.0672x over previous
import jax
import jax.numpy as jnp
from jax import lax
from jax.experimental import pallas as pl
from jax.experimental.pallas import tpu as pltpu
from jax.experimental.pallas import tpu_sc as plsc

N_GRAPHS = 2
N_LAYERS = 8
N = 10000
E = 160000
IN_NF = 256
H = 512

NC = 2
NS = 16
HS = 64
NHS = H // HS
CH = 128
NBUF = 8
EPW = 10240
NCHUNK = EPW // CH
NGROUP = NCHUNK // NBUF
EP = EPW * NS
ZROWS = 10112
RPW = ZROWS // NS
SINK = N

_HIGHEST = jax.lax.Precision.HIGHEST


def _dot3(x, w):
    xh = x.astype(jnp.bfloat16)
    xl = (x - xh.astype(jnp.float32)).astype(jnp.bfloat16)
    wh = w.astype(jnp.bfloat16)
    wl = (w - wh.astype(jnp.float32)).astype(jnp.bfloat16)
    def f(a, b):
        return jax.lax.dot_general(a, b, (((1,), (0,)), ((), ())),
                                   preferred_element_type=jnp.float32)
    return f(xh, wl) + f(xl, wh) + f(xh, wh)
_mesh = plsc.VectorSubcoreMesh(core_axis_name="c", subcore_axis_name="s")
_SC_PARAMS = pltpu.CompilerParams(use_tc_tiling_on_sc=False)


def _deg_body(dst3, out, idx_v, ones_v, zero_v, sem, deg_sp):
    c = lax.axis_index("c")
    s = lax.axis_index("s")
    half = NCHUNK // 2
    for i in range(RPW // 16 + 1):
        zero_v[pl.ds(16 * i, 16)] = jnp.zeros((16,), jnp.float32)
    for i in range(CH // 16):
        ones_v[pl.ds(16 * i, 16)] = jnp.full((16,), 1.0, jnp.float32)
    pltpu.sync_copy(zero_v.at[pl.ds(0, RPW)], deg_sp.at[pl.ds(s * RPW, RPW)])
    pltpu.sync_copy(dst3.at[s], idx_v)
    plsc.subcore_barrier()

    def chunk(j, _):
        jj = j + c * half
        pltpu.async_copy(ones_v, deg_sp.at[idx_v.at[jj]], sem, add=True).wait()
        return 0

    lax.fori_loop(0, half, chunk, 0)
    plsc.subcore_barrier()
    off = pl.multiple_of(c * ZROWS + s * RPW, 8)
    pltpu.sync_copy(deg_sp.at[pl.ds(s * RPW, RPW)], zero_v.at[pl.ds(0, RPW)])
    pltpu.sync_copy(zero_v.at[pl.ds(0, RPW)], out.at[pl.ds(off, RPW)])


def _make_deg_kernel():
    return pl.kernel(
        _deg_body,
        out_type=jax.ShapeDtypeStruct((NC * ZROWS,), jnp.float32),
        mesh=_mesh,
        scratch_types=[
            pltpu.VMEM((NCHUNK, CH), jnp.int32),
            pltpu.VMEM((CH,), jnp.float32),
            pltpu.VMEM((RPW + 16,), jnp.float32),
            pltpu.SemaphoreType.DMA,
            pltpu.VMEM_SHARED((ZROWS,), jnp.float32),
        ],
        compiler_params=_SC_PARAMS,
    )


def _scatter_body(*args):
    ys = args[:NHS]
    src3, dst3 = args[NHS], args[NHS + 1]
    zouts = args[NHS + 2:2 * NHS + 2]
    src_v, dst_v, gbuf, sem_g, sem_s, z_sp = args[2 * NHS + 2:]
    c = lax.axis_index("c")
    s = lax.axis_index("s")
    pltpu.sync_copy(src3.at[s], src_v)
    pltpu.sync_copy(dst3.at[s], dst_v)
    row0 = pl.multiple_of(s * RPW, 8)
    nlast = N - (NS - 1) * RPW

    def _stage(nrows, y, zout, to_spmem):
        rounds = []
        r = 0
        while r < nrows:
            nr = min(CH, nrows - r)
            rounds.append((len(rounds), pl.ds(row0 + r, nr), nr))
            r += nr
        for b, rows, nr in rounds:
            stg = gbuf.at[b, pl.ds(0, nr)]
            src = y.at[rows] if to_spmem else z_sp.at[rows]
            pltpu.async_copy(src, stg, sem_g[b])
        for b, rows, nr in rounds:
            stg = gbuf.at[b, pl.ds(0, nr)]
            src = y.at[rows] if to_spmem else z_sp.at[rows]
            pltpu.make_async_copy(src, stg, sem_g[b]).wait()
            dst = z_sp.at[rows] if to_spmem else zout.at[rows]
            pltpu.async_copy(stg, dst, sem_s[b])
        for b, rows, nr in rounds:
            stg = gbuf.at[b, pl.ds(0, nr)]
            dst = z_sp.at[rows] if to_spmem else zout.at[rows]
            pltpu.make_async_copy(stg, dst, sem_s[b]).wait()

    for p in range(NHS):
        y, zout = ys[p], zouts[p]

        @pl.when(c == p // (NHS // NC))
        def _pass():
            @pl.when(s < NS - 1)
            def _():
                _stage(RPW, y, zout, True)
            @pl.when(s == NS - 1)
            def _():
                _stage(nlast, y, zout, True)
            plsc.subcore_barrier()

            for b in range(NBUF):
                pltpu.async_copy(y.at[src_v.at[b]], gbuf.at[b], sem_g[b])

            def group(g, _):
                for b in range(NBUF):
                    j = g * NBUF + b
                    pltpu.make_async_copy(y.at[src_v.at[j]], gbuf.at[b],
                                          sem_g[b]).wait()
                    pltpu.async_copy(gbuf.at[b], z_sp.at[dst_v.at[j]],
                                     sem_s[b], add=True)
                for b in range(NBUF):
                    j = g * NBUF + b
                    pltpu.make_async_copy(gbuf.at[b], z_sp.at[dst_v.at[j]],
                                          sem_s[b]).wait()
                    @pl.when(g < NGROUP - 1)
                    def _():
                        pltpu.async_copy(y.at[src_v.at[j + NBUF]],
                                         gbuf.at[b], sem_g[b])
                return 0

            lax.fori_loop(0, NGROUP, group, 0)
            plsc.subcore_barrier()
            @pl.when(s < NS - 1)
            def _():
                _stage(RPW, y, zout, False)
            @pl.when(s == NS - 1)
            def _():
                _stage(nlast, y, zout, False)
            plsc.subcore_barrier()


def _make_scatter_kernel():
    return pl.kernel(
        _scatter_body,
        out_type=[jax.ShapeDtypeStruct((N, HS), jnp.float32)] * NHS,
        mesh=_mesh,
        scratch_types=[
            pltpu.VMEM((NCHUNK, CH), jnp.int32),
            pltpu.VMEM((NCHUNK, CH), jnp.int32),
            pltpu.VMEM((NBUF, CH, HS), jnp.float32),
            [pltpu.SemaphoreType.DMA] * NBUF,
            [pltpu.SemaphoreType.DMA] * NBUF,
            pltpu.VMEM_SHARED((ZROWS, HS), jnp.float32),
        ],
        compiler_params=_SC_PARAMS,
    )


_BR = 1000
_GRID = N // _BR

_ZSPEC = [pl.BlockSpec((_BR, HS), lambda r: (r, 0))] * NHS
_YSPEC = [pl.BlockSpec((_BR, HS), lambda r: (r, 0))] * NHS
_YSHAPE = [jax.ShapeDtypeStruct((N, HS), jnp.float32)] * NHS


def _emb_y_kernel(h0_ref, embw_ref, embb_ref, w_ref, dinv_ref, *y_refs):
    h = _dot3(h0_ref[...], embw_ref[...]) + embb_ref[...]
    y = dinv_ref[...] * _dot3(h, w_ref[...])
    for p in range(NHS):
        y_refs[p][...] = y[:, p * HS:(p + 1) * HS]


def _emb_y(h0, emb_W, emb_b, W0, dinv):
    return pl.pallas_call(
        _emb_y_kernel,
        grid=(_GRID,),
        in_specs=[
            pl.BlockSpec((_BR, IN_NF), lambda r: (r, 0)),
            pl.BlockSpec((IN_NF, H), lambda r: (0, 0)),
            pl.BlockSpec((1, H), lambda r: (0, 0)),
            pl.BlockSpec((H, H), lambda r: (0, 0)),
            pl.BlockSpec((_BR, 1), lambda r: (r, 0)),
        ],
        out_specs=_YSPEC,
        out_shape=_YSHAPE,
    )(h0, emb_W, emb_b, W0, dinv)


def _y_kernel(*refs):
    z_refs = refs[:NHS]
    dinv_ref, b_ref, w_ref = refs[NHS:NHS + 3]
    y_refs = refs[NHS + 3:]
    dinv = dinv_ref[...]
    z = jnp.concatenate([zr[...] for zr in z_refs], axis=1)
    h = dinv * z + b_ref[...]
    y = dinv * _dot3(h, w_ref[...])
    for p in range(NHS):
        y_refs[p][...] = y[:, p * HS:(p + 1) * HS]


def _y_step(zs, dinv, b, W):
    return pl.pallas_call(
        _y_kernel,
        grid=(_GRID,),
        in_specs=_ZSPEC + [
            pl.BlockSpec((_BR, 1), lambda r: (r, 0)),
            pl.BlockSpec((1, H), lambda r: (0, 0)),
            pl.BlockSpec((H, H), lambda r: (0, 0)),
        ],
        out_specs=_YSPEC,
        out_shape=_YSHAPE,
    )(*zs, dinv, b, W)


def _dec_kernel(*refs):
    z_refs = refs[:NHS]
    dinv_ref, b_ref, w1_ref, b1_ref, w2_ref, b2_ref, out_ref = refs[NHS:]
    dinv = dinv_ref[...]
    z = jnp.concatenate([zr[...] for zr in z_refs], axis=1)
    h = dinv * z + b_ref[...]
    t = jnp.maximum(_dot3(h, w1_ref[...]) + b1_ref[...],
                    0.0)
    o = _dot3(t, w2_ref[...]) + b2_ref[...]
    part = jnp.sum(o, axis=0, keepdims=True)

    @pl.when(pl.program_id(0) == 0)
    def _():
        out_ref[...] = jnp.zeros_like(out_ref)

    out_ref[...] += part


def _dec_sum(zs, dinv, b, W1, b1, W2, b2):
    return pl.pallas_call(
        _dec_kernel,
        grid=(_GRID,),
        in_specs=_ZSPEC + [
            pl.BlockSpec((_BR, 1), lambda r: (r, 0)),
            pl.BlockSpec((1, H), lambda r: (0, 0)),
            pl.BlockSpec((H, H), lambda r: (0, 0)),
            pl.BlockSpec((1, H), lambda r: (0, 0)),
            pl.BlockSpec((H, H), lambda r: (0, 0)),
            pl.BlockSpec((1, H), lambda r: (0, 0)),
        ],
        out_specs=pl.BlockSpec((1, H), lambda r: (0, 0)),
        out_shape=jax.ShapeDtypeStruct((1, H), jnp.float32),
    )(*zs, dinv, b, W1, b1, W2, b2)


def _gd_kernel(c_ref, w1_ref, b1_ref, w2_ref, b2_ref, out_ref):
    t = jnp.maximum(
        _dot3(c_ref[...], w1_ref[...]) + b1_ref[...],
        0.0)
    out_ref[...] = _dot3(t, w2_ref[...]) + b2_ref[...]


def kernel(h0, x, all_edges, all_edge_attr, n_nodes, emb_W, emb_b, gcn_W,
           gcn_b, dec_W1, dec_b1, dec_W2, dec_b2, gd_W1, gd_b1, gd_W2, gd_b2):
    deg_kernel = _make_deg_kernel()
    scatter_kernel = _make_scatter_kernel()
    pad_i = jnp.zeros((EP - E,), jnp.int32)
    pad_s = jnp.full((EP - E,), SINK, jnp.int32)

    sums = []
    for j in range(N_GRAPHS):
        src3 = jnp.concatenate([all_edges[j, 0], pad_i]).reshape(NS, NCHUNK, CH)
        dst3 = jnp.concatenate([all_edges[j, 1], pad_s]).reshape(NS, NCHUNK, CH)
        degp = deg_kernel(dst3)
        deg = degp[:N] + degp[ZROWS:ZROWS + N] + 1.0
        dinv = jax.lax.rsqrt(deg)[:, None]

        ys = _emb_y(h0[j], emb_W, emb_b[None, :], gcn_W[j, 0], dinv)
        for i in range(N_LAYERS):
            zs = scatter_kernel(*ys, src3, dst3)
            if i < N_LAYERS - 1:
                ys = _y_step(zs, dinv, gcn_b[j, i][None, :], gcn_W[j, i + 1])
            else:
                sums.append(_dec_sum(zs, dinv, gcn_b[j, i][None, :], dec_W1[j],
                                     dec_b1[j][None, :], dec_W2[j],
                                     dec_b2[j][None, :]))

    combined = jnp.concatenate(sums, axis=1)
    pred = pl.pallas_call(
        _gd_kernel,
        out_shape=jax.ShapeDtypeStruct((1, 1), jnp.float32),
    )(combined, gd_W1, gd_b1[None, :], gd_W2, gd_b2[None, :])
    return pred[:, 0]
